# baseline (speedup 1.0000x reference)
import functools

import jax
import jax.numpy as jnp
from jax import lax
from jax.experimental import pallas as pl
from jax.experimental.pallas import tpu as pltpu
from jax.experimental.pallas import tpu_sc as plsc

D = 64
N = 4096 * 200
LPC = 128
NW = 32
CPW = N // (LPC * NW)
SCALE = 8.0

_mesh = plsc.VectorSubcoreMesh(core_axis_name="c", subcore_axis_name="s")


@functools.partial(
    pl.kernel,
    out_type=jax.ShapeDtypeStruct((N, D), jnp.float32),
    mesh=_mesh,
    compiler_params=pltpu.CompilerParams(use_tc_tiling_on_sc=False),
    scratch_types=[
        pltpu.VMEM((CPW, LPC), jnp.int32),
        pltpu.VMEM((LPC, D), jnp.float32),
        pltpu.SemaphoreType.DMA,
    ],
)
def _emb_lookup(x_hbm, table_hbm, out_hbm, idx_v, rows_v, sem):
    wid = lax.axis_index("s") * 2 + lax.axis_index("c")
    pltpu.sync_copy(x_hbm.at[pl.ds(wid * CPW, CPW)], idx_v)

    def chunk(i, carry):
        pltpu.async_copy(table_hbm.at[idx_v.at[i]], rows_v, sem).wait()

        def srow(r, c2):
            for cc in range(D // 16):
                sl = (r, pl.ds(cc * 16, 16))
                rows_v[sl] = rows_v[sl] * SCALE
            return c2

        lax.fori_loop(0, LPC, srow, 0)
        base = (wid * CPW + i) * LPC
        pltpu.sync_copy(rows_v, out_hbm.at[pl.ds(base, LPC)])
        return carry

    lax.fori_loop(0, CPW, chunk, 0)


def kernel(x, table):
    x2 = x.reshape(N // LPC, LPC)
    out = _emb_lookup(x2, table)
    return out.reshape(4096, 200, D)
